# baseline (speedup 1.0000x reference)
import functools

import jax
import jax.numpy as jnp
from jax.experimental import pallas as pl

_HEADS = 8
_DIM = 32
_HIDDEN = 256
_NEG_SLOPE = 0.2
_ROW_BLOCK = 1000


def _proj_body(x_ref, w_ref, b_ref, ls_ref, ld_ref, h_ref, a_ref):
    h = jnp.dot(x_ref[...], w_ref[...], preferred_element_type=jnp.float32)
    h = h + b_ref[...]
    h_ref[...] = h
    hr = h.reshape(h.shape[0], _HEADS, _DIM)
    a_s = (hr * ls_ref[...][None]).sum(-1)
    a_d = (hr * ld_ref[...][None]).sum(-1)
    a_ref[...] = jnp.concatenate([a_s, a_d], axis=1)


def _project(x, w, b, lin_src, lin_dst):
    n, din = x.shape
    grid = (n // _ROW_BLOCK,)
    h, a = pl.pallas_call(
        _proj_body,
        grid=grid,
        in_specs=[
            pl.BlockSpec((_ROW_BLOCK, din), lambda i: (i, 0)),
            pl.BlockSpec((din, _HIDDEN), lambda i: (0, 0)),
            pl.BlockSpec((1, _HIDDEN), lambda i: (0, 0)),
            pl.BlockSpec((_HEADS, _DIM), lambda i: (0, 0)),
            pl.BlockSpec((_HEADS, _DIM), lambda i: (0, 0)),
        ],
        out_specs=[
            pl.BlockSpec((_ROW_BLOCK, _HIDDEN), lambda i: (i, 0)),
            pl.BlockSpec((_ROW_BLOCK, 2 * _HEADS), lambda i: (i, 0)),
        ],
        out_shape=[
            jax.ShapeDtypeStruct((n, _HIDDEN), jnp.float32),
            jax.ShapeDtypeStruct((n, 2 * _HEADS), jnp.float32),
        ],
    )(x, w, b.reshape(1, _HIDDEN), lin_src, lin_dst)
    return h, a


def _edge_phase(h_src, a_src, a_dst, ei, num_dst):
    s, d = ei[0], ei[1]
    alpha = a_src[s] + a_dst[d]
    alpha = jnp.maximum(alpha, _NEG_SLOPE * alpha)
    amax = jax.ops.segment_max(alpha, d, num_segments=num_dst)
    amax = jnp.where(jnp.isfinite(amax), amax, 0.0)
    ex = jnp.exp(alpha - amax[d])
    denom = jax.ops.segment_sum(ex, d, num_segments=num_dst)
    alpha_n = ex / (denom[d] + 1e-16)
    msg = h_src.reshape(-1, _HEADS, _DIM)[s] * alpha_n[:, :, None]
    out = jax.ops.segment_sum(msg, d, num_segments=num_dst).reshape(-1, _HIDDEN)
    return jax.nn.relu(out)


def kernel(x_author, x_paper, edge_index_writes, edge_index_written_by, params):
    xa, xp = x_author, x_paper
    for p in params:
        ha, aa = _project(
            xa, p["proj_w"]["author"], p["proj_b"]["author"],
            p["lin_src"]["author__writes__paper"],
            p["lin_dst"]["paper__written_by__author"],
        )
        hp, ap = _project(
            xp, p["proj_w"]["paper"], p["proj_b"]["paper"],
            p["lin_src"]["paper__written_by__author"],
            p["lin_dst"]["author__writes__paper"],
        )
        out_paper = _edge_phase(
            ha, aa[:, :_HEADS], ap[:, _HEADS:], edge_index_writes, hp.shape[0]
        )
        out_author = _edge_phase(
            hp, ap[:, :_HEADS], aa[:, _HEADS:], edge_index_written_by, ha.shape[0]
        )
        xa, xp = out_author, out_paper
    return xa, xp


# speedup vs baseline: 30.5409x; 30.5409x over previous
import dataclasses
import functools

import jax
import jax.numpy as jnp
from jax import lax
from jax.experimental import pallas as pl
from jax.experimental.pallas import tpu as pltpu
from jax.experimental.pallas import tpu_sc as plsc

_HEADS = 8
_DIM = 32
_HIDDEN = 256
_NEG_SLOPE = 0.2
_ROW_BLOCK = 1000
_C = 400
_NSUB = 16
_EPS = 1e-16


def _att_coefs(h, ls, ld):
    hr = h.reshape(h.shape[0], _HEADS, _DIM)
    return (hr * ls[None]).sum(-1), (hr * ld[None]).sum(-1)


def _proj_body(x_ref, w_ref, b_ref, ls_ref, ld_ref, h4_ref, as_ref, ad_ref):
    h = jnp.dot(x_ref[...], w_ref[...], preferred_element_type=jnp.float32)
    h = h + b_ref[...]
    as_ref[...], ad_ref[...] = _att_coefs(h, ls_ref[...], ld_ref[...])
    h4_ref[...] = jnp.swapaxes(h.reshape(h.shape[0], 4, 64), 0, 1)


def _unnormalize(acc_ref, den_ref):
    rb = acc_ref.shape[1]
    xs = []
    for q in range(4):
        dpair = den_ref[q // 2][:, (q % 2) * 2:(q % 2) * 2 + 2]
        scale = 1.0 / (dpair + _EPS)
        scale = jnp.broadcast_to(scale[:, :, None], (rb, 2, _DIM))
        xs.append(jax.nn.relu(acc_ref[q]) * scale.reshape(rb, 64))
    return jnp.concatenate(xs, axis=1)


def _norm_proj_body(acc_ref, den_ref, w_ref, b_ref, ls_ref, ld_ref,
                    h4_ref, as_ref, ad_ref):
    x = _unnormalize(acc_ref, den_ref)
    h = jnp.dot(x, w_ref[...], preferred_element_type=jnp.float32)
    h = h + b_ref[...]
    as_ref[...], ad_ref[...] = _att_coefs(h, ls_ref[...], ld_ref[...])
    h4_ref[...] = jnp.swapaxes(h.reshape(x.shape[0], 4, 64), 0, 1)


def _final_body(acc_ref, den_ref, o_ref):
    o_ref[...] = _unnormalize(acc_ref, den_ref)


def _project(x_or_accden, w, b, lin_src, lin_dst, *, fused_norm):
    n = 10000
    grid = (n // _ROW_BLOCK,)
    out_specs = [
        pl.BlockSpec((4, _ROW_BLOCK, 64), lambda i: (0, i, 0)),
        pl.BlockSpec((_ROW_BLOCK, _HEADS), lambda i: (i, 0)),
        pl.BlockSpec((_ROW_BLOCK, _HEADS), lambda i: (i, 0)),
    ]
    out_shape = [
        jax.ShapeDtypeStruct((4, n, 64), jnp.float32),
        jax.ShapeDtypeStruct((n, _HEADS), jnp.float32),
        jax.ShapeDtypeStruct((n, _HEADS), jnp.float32),
    ]
    w_specs = [
        pl.BlockSpec((w.shape[0], _HIDDEN), lambda i: (0, 0)),
        pl.BlockSpec((1, _HIDDEN), lambda i: (0, 0)),
        pl.BlockSpec((_HEADS, _DIM), lambda i: (0, 0)),
        pl.BlockSpec((_HEADS, _DIM), lambda i: (0, 0)),
    ]
    if fused_norm:
        acc, den = x_or_accden
        h4, a_s, a_d = pl.pallas_call(
            _norm_proj_body,
            grid=grid,
            in_specs=[
                pl.BlockSpec((4, _ROW_BLOCK, 64), lambda i: (0, i, 0)),
                pl.BlockSpec((2, _ROW_BLOCK, 16), lambda i: (0, i, 0)),
            ] + w_specs,
            out_specs=out_specs,
            out_shape=out_shape,
        )(acc.reshape(4, n, 64), den.reshape(2, n, 16),
          w, b.reshape(1, _HIDDEN), lin_src, lin_dst)
    else:
        x = x_or_accden
        h4, a_s, a_d = pl.pallas_call(
            _proj_body,
            grid=grid,
            in_specs=[
                pl.BlockSpec((_ROW_BLOCK, x.shape[1]), lambda i: (i, 0)),
            ] + w_specs,
            out_specs=out_specs,
            out_shape=out_shape,
        )(x, w, b.reshape(1, _HIDDEN), lin_src, lin_dst)
    return h4.reshape(4 * n, 64), a_s, a_d


def _finalize(acc, den):
    n = 10000
    grid = (n // _ROW_BLOCK,)
    return pl.pallas_call(
        _final_body,
        grid=grid,
        in_specs=[
            pl.BlockSpec((4, _ROW_BLOCK, 64), lambda i: (0, i, 0)),
            pl.BlockSpec((2, _ROW_BLOCK, 16), lambda i: (0, i, 0)),
        ],
        out_specs=pl.BlockSpec((_ROW_BLOCK, _HIDDEN), lambda i: (i, 0)),
        out_shape=jax.ShapeDtypeStruct((n, _HIDDEN), jnp.float32),
    )(acc.reshape(4, n, 64), den.reshape(2, n, 16))


def _sc_edge(h4, a_src, a_dst, s_idx, d_idx):
    n = a_src.shape[0]
    e = s_idx.shape[0]
    per_sub = e // _NSUB
    n_chunks = per_sub // _C

    mesh = plsc.VectorSubcoreMesh(core_axis_name="c", subcore_axis_name="s")
    cp = pltpu.CompilerParams()
    if "needs_layout_passes" in pltpu.CompilerParams.__dataclass_fields__:
        cp = dataclasses.replace(cp, needs_layout_passes=False)
    if "use_tc_tiling_on_sc" in pltpu.CompilerParams.__dataclass_fields__:
        cp = dataclasses.replace(cp, use_tc_tiling_on_sc=False)

    @functools.partial(
        pl.kernel,
        compiler_params=cp,
        out_type=[
            jax.ShapeDtypeStruct((4 * n, 64), jnp.float32),
            jax.ShapeDtypeStruct((2 * n * 16,), jnp.float32),
        ],
        mesh=mesh,
        scratch_types=[
            pltpu.VMEM((_C,), jnp.int32),
            pltpu.VMEM((_C,), jnp.int32),
            pltpu.VMEM((_C,), jnp.int32),
            pltpu.VMEM((2 * _C,), jnp.int32),
            pltpu.VMEM((_C, 8), jnp.float32),
            pltpu.VMEM((_C, 8), jnp.float32),
            pltpu.VMEM((2 * _C,), jnp.float32),
            pltpu.VMEM((_C, 64), jnp.float32),
            pltpu.VMEM((200, 64), jnp.float32),
            pltpu.VMEM((16 * (10000 // 16),), jnp.float32),
            pltpu.VMEM_SHARED((n, 64), jnp.float32),
            pltpu.VMEM_SHARED((n * 16,), jnp.float32),
        ],
    )
    def k(h_hbm, as_hbm, ad_hbm, s_hbm, d_hbm, acc_hbm, den_hbm,
          sidx_v, didx_v, s2_v, didx16_v, as_v, ad_v, ex_v, h_v,
          zb_v, zd_v, acc_sh, den_sh):
        cid = lax.axis_index("c")
        sid = lax.axis_index("s")
        zeros16 = jnp.full((16,), 0.0, jnp.float32)
        rpd = 10000 // 16

        @pl.loop(0, 200)
        def _(r):
            for t in range(4):
                zb_v[r, pl.ds(t * 16, 16)] = zeros16

        @pl.loop(0, rpd)
        def _(t):
            zd_v[pl.ds(t * 16, 16)] = zeros16

        def zero_acc():
            @pl.when(sid < 10)
            def _():
                for t in range(5):
                    pltpu.sync_copy(
                        zb_v, acc_sh.at[pl.ds(sid * 1000 + t * 200, 200), :])

        zero_acc()
        pltpu.sync_copy(zd_v, den_sh.at[pl.ds(sid * 16 * rpd, 16 * rpd)])
        plsc.subcore_barrier()

        lane = lax.iota(jnp.int32, 16)
        rloc = lane // 2
        cloc = lane - rloc * 2

        for qq in range(2):
            qg = cid * 2 + qq
            row_off = qg * n

            @pl.loop(0, n_chunks)
            def _(kk):
                base = sid * per_sub + kk * _C
                pltpu.sync_copy(s_hbm.at[pl.ds(base, _C)], sidx_v)
                pltpu.sync_copy(d_hbm.at[pl.ds(base, _C)], didx_v)

                @pl.loop(0, _C // 16)
                def _(t):
                    s2_v[pl.ds(t * 16, 16)] = sidx_v[pl.ds(t * 16, 16)] + row_off

                pltpu.sync_copy(as_hbm.at[sidx_v], as_v)
                pltpu.sync_copy(ad_hbm.at[didx_v], ad_v)

                @pl.loop(0, _C // 8)
                def _(q):
                    r = q * 8 + rloc
                    c = cloc + qg * 2
                    al = (plsc.load_gather(as_v, [r, c])
                          + plsc.load_gather(ad_v, [r, c]))
                    al = jnp.maximum(al, _NEG_SLOPE * al)
                    ex_v[pl.ds(q * 16, 16)] = jnp.exp(al)
                    dv = plsc.load_gather(didx_v, [r])
                    didx16_v[pl.ds(q * 16, 16)] = dv * 16 + (cloc + qq * 2)

                pltpu.sync_copy(h_hbm.at[s2_v], h_v)

                @pl.loop(0, _C)
                def _(i):
                    for j2 in range(2):
                        idxv = jnp.full((16,), 0, jnp.int32) + (i * 2 + j2)
                        sc = plsc.load_gather(ex_v, [idxv])
                        for half in range(2):
                            sl = pl.ds(j2 * 32 + half * 16, 16)
                            h_v[i, sl] = h_v[i, sl] * sc

                pltpu.sync_copy(ex_v, den_sh.at[didx16_v], add=True)
                pltpu.sync_copy(h_v, acc_sh.at[didx_v], add=True)

            plsc.subcore_barrier()

            @pl.when(sid < 10)
            def _():
                pltpu.sync_copy(acc_sh.at[pl.ds(sid * 1000, 1000), :],
                                acc_hbm.at[pl.ds(row_off + sid * 1000, 1000), :])

            if qq == 0:
                zero_acc()
                plsc.subcore_barrier()

        pltpu.sync_copy(den_sh.at[pl.ds(sid * 16 * rpd, 16 * rpd)],
                        den_hbm.at[pl.ds(cid * n * 16 + sid * 16 * rpd, 16 * rpd)])

    acc, den = k(h4, a_src, a_dst, s_idx, d_idx)
    return acc, den


def kernel(x_author, x_paper, edge_index_writes, edge_index_written_by, params):
    sw, dw = edge_index_writes[0], edge_index_writes[1]
    swb, dwb = edge_index_written_by[0], edge_index_written_by[1]

    state_a = x_author
    state_p = x_paper
    for l, p in enumerate(params):
        fused = l > 0
        ha4, a_s_w, a_d_wb = _project(
            state_a, p["proj_w"]["author"], p["proj_b"]["author"],
            p["lin_src"]["author__writes__paper"],
            p["lin_dst"]["paper__written_by__author"],
            fused_norm=fused,
        )
        hp4, a_s_wb, a_d_w = _project(
            state_p, p["proj_w"]["paper"], p["proj_b"]["paper"],
            p["lin_src"]["paper__written_by__author"],
            p["lin_dst"]["author__writes__paper"],
            fused_norm=fused,
        )
        acc_p, den_p = _sc_edge(ha4, a_s_w, a_d_w, sw, dw)
        acc_a, den_a = _sc_edge(hp4, a_s_wb, a_d_wb, swb, dwb)
        state_a = (acc_a, den_a)
        state_p = (acc_p, den_p)

    out_a = _finalize(*state_a)
    out_p = _finalize(*state_p)
    return out_a, out_p


# speedup vs baseline: 34.1329x; 1.1176x over previous
import dataclasses
import functools

import jax
import jax.numpy as jnp
from jax import lax
from jax.experimental import pallas as pl
from jax.experimental.pallas import tpu as pltpu
from jax.experimental.pallas import tpu_sc as plsc

_HEADS = 8
_DIM = 32
_HIDDEN = 256
_NEG_SLOPE = 0.2
_ROW_BLOCK = 1000
_C = 400
_NSUB = 16
_EPS = 1e-16


def _att_coefs(h, ls, ld):
    hr = h.reshape(h.shape[0], _HEADS, _DIM)
    return (hr * ls[None]).sum(-1), (hr * ld[None]).sum(-1)


def _proj_body(x_ref, w_ref, b_ref, ls_ref, ld_ref, h4_ref, as_ref, ad_ref):
    h = jnp.dot(x_ref[...], w_ref[...], preferred_element_type=jnp.float32)
    h = h + b_ref[...]
    as_ref[...], ad_ref[...] = _att_coefs(h, ls_ref[...], ld_ref[...])
    h4_ref[...] = jnp.swapaxes(h.reshape(h.shape[0], 4, 64), 0, 1)


def _unnormalize(acc_ref, den_ref):
    rb = acc_ref.shape[1]
    xs = []
    for q in range(4):
        dpair = den_ref[q // 2][:, (q % 2) * 2:(q % 2) * 2 + 2]
        scale = 1.0 / (dpair + _EPS)
        scale = jnp.broadcast_to(scale[:, :, None], (rb, 2, _DIM))
        xs.append(jax.nn.relu(acc_ref[q]) * scale.reshape(rb, 64))
    return jnp.concatenate(xs, axis=1)


def _norm_proj_body(acc_ref, den_ref, w_ref, b_ref, ls_ref, ld_ref,
                    h4_ref, as_ref, ad_ref):
    x = _unnormalize(acc_ref, den_ref)
    h = jnp.dot(x, w_ref[...], preferred_element_type=jnp.float32)
    h = h + b_ref[...]
    as_ref[...], ad_ref[...] = _att_coefs(h, ls_ref[...], ld_ref[...])
    h4_ref[...] = jnp.swapaxes(h.reshape(x.shape[0], 4, 64), 0, 1)


def _final_body(acc_ref, den_ref, o_ref):
    o_ref[...] = _unnormalize(acc_ref, den_ref)


def _project(x_or_accden, w, b, lin_src, lin_dst, *, fused_norm):
    n = 10000
    grid = (n // _ROW_BLOCK,)
    out_specs = [
        pl.BlockSpec((4, _ROW_BLOCK, 64), lambda i: (0, i, 0)),
        pl.BlockSpec((_ROW_BLOCK, _HEADS), lambda i: (i, 0)),
        pl.BlockSpec((_ROW_BLOCK, _HEADS), lambda i: (i, 0)),
    ]
    out_shape = [
        jax.ShapeDtypeStruct((4, n, 64), jnp.float32),
        jax.ShapeDtypeStruct((n, _HEADS), jnp.float32),
        jax.ShapeDtypeStruct((n, _HEADS), jnp.float32),
    ]
    w_specs = [
        pl.BlockSpec((w.shape[0], _HIDDEN), lambda i: (0, 0)),
        pl.BlockSpec((1, _HIDDEN), lambda i: (0, 0)),
        pl.BlockSpec((_HEADS, _DIM), lambda i: (0, 0)),
        pl.BlockSpec((_HEADS, _DIM), lambda i: (0, 0)),
    ]
    if fused_norm:
        acc, den = x_or_accden
        h4, a_s, a_d = pl.pallas_call(
            _norm_proj_body,
            grid=grid,
            in_specs=[
                pl.BlockSpec((4, _ROW_BLOCK, 64), lambda i: (0, i, 0)),
                pl.BlockSpec((2, _ROW_BLOCK, 16), lambda i: (0, i, 0)),
            ] + w_specs,
            out_specs=out_specs,
            out_shape=out_shape,
        )(acc.reshape(4, n, 64), den.reshape(2, n, 16),
          w, b.reshape(1, _HIDDEN), lin_src, lin_dst)
    else:
        x = x_or_accden
        h4, a_s, a_d = pl.pallas_call(
            _proj_body,
            grid=grid,
            in_specs=[
                pl.BlockSpec((_ROW_BLOCK, x.shape[1]), lambda i: (i, 0)),
            ] + w_specs,
            out_specs=out_specs,
            out_shape=out_shape,
        )(x, w, b.reshape(1, _HIDDEN), lin_src, lin_dst)
    return h4.reshape(4 * n, 64), a_s, a_d


def _finalize(acc, den):
    n = 10000
    grid = (n // _ROW_BLOCK,)
    return pl.pallas_call(
        _final_body,
        grid=grid,
        in_specs=[
            pl.BlockSpec((4, _ROW_BLOCK, 64), lambda i: (0, i, 0)),
            pl.BlockSpec((2, _ROW_BLOCK, 16), lambda i: (0, i, 0)),
        ],
        out_specs=pl.BlockSpec((_ROW_BLOCK, _HIDDEN), lambda i: (i, 0)),
        out_shape=jax.ShapeDtypeStruct((n, _HIDDEN), jnp.float32),
    )(acc.reshape(4, n, 64), den.reshape(2, n, 16))


def _sc_edge(h4, a_src, a_dst, s_idx, d_idx):
    n = a_src.shape[0]
    e = s_idx.shape[0]
    per_sub = e // _NSUB
    n_chunks = per_sub // _C

    mesh = plsc.VectorSubcoreMesh(core_axis_name="c", subcore_axis_name="s")
    cp = pltpu.CompilerParams()
    if "needs_layout_passes" in pltpu.CompilerParams.__dataclass_fields__:
        cp = dataclasses.replace(cp, needs_layout_passes=False)
    if "use_tc_tiling_on_sc" in pltpu.CompilerParams.__dataclass_fields__:
        cp = dataclasses.replace(cp, use_tc_tiling_on_sc=False)

    @functools.partial(
        pl.kernel,
        compiler_params=cp,
        out_type=[
            jax.ShapeDtypeStruct((4 * n, 64), jnp.float32),
            jax.ShapeDtypeStruct((2 * n * 16,), jnp.float32),
        ],
        mesh=mesh,
        scratch_types=[
            pltpu.VMEM((_C,), jnp.int32),
            pltpu.VMEM((_C,), jnp.int32),
            pltpu.VMEM((_C,), jnp.int32),
            pltpu.VMEM((2 * _C,), jnp.int32),
            pltpu.VMEM((_C, 8), jnp.float32),
            pltpu.VMEM((_C, 8), jnp.float32),
            pltpu.VMEM((2 * _C,), jnp.float32),
            pltpu.VMEM((_C, 64), jnp.float32),
            pltpu.VMEM((200, 64), jnp.float32),
            pltpu.VMEM((16 * (10000 // 16),), jnp.float32),
            pltpu.VMEM_SHARED((n, 64), jnp.float32),
            pltpu.VMEM_SHARED((n * 16,), jnp.float32),
            pltpu.SemaphoreType.DMA,
        ],
    )
    def k(h_hbm, as_hbm, ad_hbm, s_hbm, d_hbm, acc_hbm, den_hbm,
          sidx_v, didx_v, s2_v, didx16_v, as_v, ad_v, ex_v, h_v,
          zb_v, zd_v, acc_sh, den_sh, hg_sem):
        cid = lax.axis_index("c")
        sid = lax.axis_index("s")
        zeros16 = jnp.full((16,), 0.0, jnp.float32)
        rpd = 10000 // 16

        @pl.loop(0, 200)
        def _(r):
            for t in range(4):
                zb_v[r, pl.ds(t * 16, 16)] = zeros16

        @pl.loop(0, rpd)
        def _(t):
            zd_v[pl.ds(t * 16, 16)] = zeros16

        def zero_acc():
            @pl.when(sid < 10)
            def _():
                for t in range(5):
                    pltpu.sync_copy(
                        zb_v, acc_sh.at[pl.ds(sid * 1000 + t * 200, 200), :])

        zero_acc()
        pltpu.sync_copy(zd_v, den_sh.at[pl.ds(sid * 16 * rpd, 16 * rpd)])
        plsc.subcore_barrier()

        lane = lax.iota(jnp.int32, 16)
        rloc = lane // 2
        cloc = lane - rloc * 2

        for qq in range(2):
            qg = cid * 2 + qq
            row_off = qg * n

            @pl.loop(0, n_chunks)
            def _(kk):
                base = sid * per_sub + kk * _C
                pltpu.sync_copy(s_hbm.at[pl.ds(base, _C)], sidx_v)
                pltpu.sync_copy(d_hbm.at[pl.ds(base, _C)], didx_v)

                @pl.loop(0, _C // 16)
                def _(t):
                    s2_v[pl.ds(t * 16, 16)] = sidx_v[pl.ds(t * 16, 16)] + row_off

                pltpu.async_copy(h_hbm.at[s2_v], h_v, hg_sem)
                pltpu.sync_copy(as_hbm.at[sidx_v], as_v)
                pltpu.sync_copy(ad_hbm.at[didx_v], ad_v)

                @pl.loop(0, _C // 8)
                def _(q):
                    r = q * 8 + rloc
                    c = cloc + qg * 2
                    al = (plsc.load_gather(as_v, [r, c])
                          + plsc.load_gather(ad_v, [r, c]))
                    al = jnp.maximum(al, _NEG_SLOPE * al)
                    ex_v[pl.ds(q * 16, 16)] = jnp.exp(al)
                    dv = plsc.load_gather(didx_v, [r])
                    didx16_v[pl.ds(q * 16, 16)] = dv * 16 + (cloc + qq * 2)

                pltpu.make_async_copy(h_hbm.at[s2_v], h_v, hg_sem).wait()

                @pl.loop(0, _C)
                def _(i):
                    for j2 in range(2):
                        idxv = jnp.full((16,), 0, jnp.int32) + (i * 2 + j2)
                        sc = plsc.load_gather(ex_v, [idxv])
                        for half in range(2):
                            sl = pl.ds(j2 * 32 + half * 16, 16)
                            h_v[i, sl] = h_v[i, sl] * sc

                pltpu.sync_copy(ex_v, den_sh.at[didx16_v], add=True)
                pltpu.sync_copy(h_v, acc_sh.at[didx_v], add=True)

            plsc.subcore_barrier()

            @pl.when(sid < 10)
            def _():
                pltpu.sync_copy(acc_sh.at[pl.ds(sid * 1000, 1000), :],
                                acc_hbm.at[pl.ds(row_off + sid * 1000, 1000), :])

            if qq == 0:
                zero_acc()
                plsc.subcore_barrier()

        pltpu.sync_copy(den_sh.at[pl.ds(sid * 16 * rpd, 16 * rpd)],
                        den_hbm.at[pl.ds(cid * n * 16 + sid * 16 * rpd, 16 * rpd)])

    acc, den = k(h4, a_src, a_dst, s_idx, d_idx)
    return acc, den


def kernel(x_author, x_paper, edge_index_writes, edge_index_written_by, params):
    sw, dw = edge_index_writes[0], edge_index_writes[1]
    swb, dwb = edge_index_written_by[0], edge_index_written_by[1]

    state_a = x_author
    state_p = x_paper
    for l, p in enumerate(params):
        fused = l > 0
        ha4, a_s_w, a_d_wb = _project(
            state_a, p["proj_w"]["author"], p["proj_b"]["author"],
            p["lin_src"]["author__writes__paper"],
            p["lin_dst"]["paper__written_by__author"],
            fused_norm=fused,
        )
        hp4, a_s_wb, a_d_w = _project(
            state_p, p["proj_w"]["paper"], p["proj_b"]["paper"],
            p["lin_src"]["paper__written_by__author"],
            p["lin_dst"]["author__writes__paper"],
            fused_norm=fused,
        )
        acc_p, den_p = _sc_edge(ha4, a_s_w, a_d_w, sw, dw)
        acc_a, den_a = _sc_edge(hp4, a_s_wb, a_d_wb, swb, dwb)
        state_a = (acc_a, den_a)
        state_p = (acc_p, den_p)

    out_a = _finalize(*state_a)
    out_p = _finalize(*state_p)
    return out_a, out_p


# speedup vs baseline: 37.9848x; 1.1129x over previous
import dataclasses
import functools

import jax
import jax.numpy as jnp
from jax import lax
from jax.experimental import pallas as pl
from jax.experimental.pallas import tpu as pltpu
from jax.experimental.pallas import tpu_sc as plsc

_HEADS = 8
_DIM = 32
_HIDDEN = 256
_NEG_SLOPE = 0.2
_ROW_BLOCK = 1000
_C = 400
_NSUB = 16
_EPS = 1e-16


def _att_coefs(h, ls, ld):
    hr = h.reshape(h.shape[0], _HEADS, _DIM)
    return (hr * ls[None]).sum(-1), (hr * ld[None]).sum(-1)


def _proj_body(x_ref, w_ref, b_ref, ls_ref, ld_ref, h4_ref, as_ref, ad_ref):
    h = jnp.dot(x_ref[...], w_ref[...], preferred_element_type=jnp.float32)
    h = h + b_ref[...]
    as_ref[...], ad_ref[...] = _att_coefs(h, ls_ref[...], ld_ref[...])
    h4_ref[...] = jnp.swapaxes(h.reshape(h.shape[0], 4, 64), 0, 1)


def _unnormalize(acc_ref, den_ref):
    rb = acc_ref.shape[1]
    xs = []
    for q in range(4):
        dpair = den_ref[q // 2][:, (q % 2) * 2:(q % 2) * 2 + 2]
        scale = 1.0 / (dpair + _EPS)
        scale = jnp.broadcast_to(scale[:, :, None], (rb, 2, _DIM))
        xs.append(jax.nn.relu(acc_ref[q]) * scale.reshape(rb, 64))
    return jnp.concatenate(xs, axis=1)


def _norm_proj_body(acc_ref, den_ref, w_ref, b_ref, ls_ref, ld_ref,
                    h4_ref, as_ref, ad_ref):
    x = _unnormalize(acc_ref, den_ref)
    h = jnp.dot(x, w_ref[...], preferred_element_type=jnp.float32)
    h = h + b_ref[...]
    as_ref[...], ad_ref[...] = _att_coefs(h, ls_ref[...], ld_ref[...])
    h4_ref[...] = jnp.swapaxes(h.reshape(x.shape[0], 4, 64), 0, 1)


def _final_body(acc_ref, den_ref, o_ref):
    o_ref[...] = _unnormalize(acc_ref, den_ref)


def _project(x_or_accden, w, b, lin_src, lin_dst, *, fused_norm):
    n = 10000
    grid = (n // _ROW_BLOCK,)
    out_specs = [
        pl.BlockSpec((4, _ROW_BLOCK, 64), lambda i: (0, i, 0)),
        pl.BlockSpec((_ROW_BLOCK, _HEADS), lambda i: (i, 0)),
        pl.BlockSpec((_ROW_BLOCK, _HEADS), lambda i: (i, 0)),
    ]
    out_shape = [
        jax.ShapeDtypeStruct((4, n, 64), jnp.float32),
        jax.ShapeDtypeStruct((n, _HEADS), jnp.float32),
        jax.ShapeDtypeStruct((n, _HEADS), jnp.float32),
    ]
    w_specs = [
        pl.BlockSpec((w.shape[0], _HIDDEN), lambda i: (0, 0)),
        pl.BlockSpec((1, _HIDDEN), lambda i: (0, 0)),
        pl.BlockSpec((_HEADS, _DIM), lambda i: (0, 0)),
        pl.BlockSpec((_HEADS, _DIM), lambda i: (0, 0)),
    ]
    if fused_norm:
        acc, den = x_or_accden
        h4, a_s, a_d = pl.pallas_call(
            _norm_proj_body,
            grid=grid,
            in_specs=[
                pl.BlockSpec((4, _ROW_BLOCK, 64), lambda i: (0, i, 0)),
                pl.BlockSpec((2, _ROW_BLOCK, 8), lambda i: (0, i, 0)),
            ] + w_specs,
            out_specs=out_specs,
            out_shape=out_shape,
        )(acc.reshape(4, n, 64), den.reshape(2, n, 8),
          w, b.reshape(1, _HIDDEN), lin_src, lin_dst)
    else:
        x = x_or_accden
        h4, a_s, a_d = pl.pallas_call(
            _proj_body,
            grid=grid,
            in_specs=[
                pl.BlockSpec((_ROW_BLOCK, x.shape[1]), lambda i: (i, 0)),
            ] + w_specs,
            out_specs=out_specs,
            out_shape=out_shape,
        )(x, w, b.reshape(1, _HIDDEN), lin_src, lin_dst)
    return h4.reshape(4 * n, 64), a_s, a_d


def _finalize(acc, den):
    n = 10000
    grid = (n // _ROW_BLOCK,)
    return pl.pallas_call(
        _final_body,
        grid=grid,
        in_specs=[
            pl.BlockSpec((4, _ROW_BLOCK, 64), lambda i: (0, i, 0)),
            pl.BlockSpec((2, _ROW_BLOCK, 8), lambda i: (0, i, 0)),
        ],
        out_specs=pl.BlockSpec((_ROW_BLOCK, _HIDDEN), lambda i: (i, 0)),
        out_shape=jax.ShapeDtypeStruct((n, _HIDDEN), jnp.float32),
    )(acc.reshape(4, n, 64), den.reshape(2, n, 8))


def _sc_edge(h4, a_src, a_dst, s_idx, d_idx):
    n = a_src.shape[0]
    e = s_idx.shape[0]
    per_sub = e // _NSUB
    n_chunks = per_sub // _C

    mesh = plsc.VectorSubcoreMesh(core_axis_name="c", subcore_axis_name="s")
    cp = pltpu.CompilerParams()
    if "needs_layout_passes" in pltpu.CompilerParams.__dataclass_fields__:
        cp = dataclasses.replace(cp, needs_layout_passes=False)
    if "use_tc_tiling_on_sc" in pltpu.CompilerParams.__dataclass_fields__:
        cp = dataclasses.replace(cp, use_tc_tiling_on_sc=False)

    @functools.partial(
        pl.kernel,
        compiler_params=cp,
        out_type=[
            jax.ShapeDtypeStruct((4 * n, 64), jnp.float32),
            jax.ShapeDtypeStruct((2 * n * 8,), jnp.float32),
        ],
        mesh=mesh,
        scratch_types=[
            pltpu.VMEM((_C,), jnp.int32),
            pltpu.VMEM((_C,), jnp.int32),
            pltpu.VMEM((_C,), jnp.int32),
            pltpu.VMEM((_C,), jnp.int32),
            pltpu.VMEM((2 * _C,), jnp.int32),
            pltpu.VMEM((2 * _C,), jnp.int32),
            pltpu.VMEM((_C, 8), jnp.float32),
            pltpu.VMEM((_C, 8), jnp.float32),
            pltpu.VMEM((2 * _C,), jnp.float32),
            pltpu.VMEM((2 * _C,), jnp.float32),
            pltpu.VMEM((_C, 64), jnp.float32),
            pltpu.VMEM((_C, 64), jnp.float32),
            pltpu.VMEM((200, 64), jnp.float32),
            pltpu.VMEM((5120,), jnp.float32),
            pltpu.VMEM_SHARED((n, 64), jnp.float32),
            pltpu.VMEM_SHARED((n * 8,), jnp.float32),
            pltpu.SemaphoreType.DMA,
            pltpu.SemaphoreType.DMA,
            pltpu.SemaphoreType.DMA,
            pltpu.SemaphoreType.DMA,
            pltpu.SemaphoreType.DMA,
        ],
    )
    def k(h_hbm, as_hbm, ad_hbm, s_hbm, d_hbm, acc_hbm, den_hbm,
          sidx_v, didx_v0, didx_v1, s2_v, didx16_v0, didx16_v1,
          as_v, ad_v, ex_v0, ex_v1, h_v0, h_v1,
          zb_v, zd_v, acc_sh, den_sh, hg_sem, es0, es1, hs0, hs1):
        didx_b = [didx_v0, didx_v1]
        didx16_b = [didx16_v0, didx16_v1]
        ex_b = [ex_v0, ex_v1]
        h_b = [h_v0, h_v1]
        es_b = [es0, es1]
        hs_b = [hs0, hs1]
        cid = lax.axis_index("c")
        sid = lax.axis_index("s")
        zeros16 = jnp.full((16,), 0.0, jnp.float32)
        @pl.loop(0, 200)
        def _(r):
            for t in range(4):
                zb_v[r, pl.ds(t * 16, 16)] = zeros16

        @pl.loop(0, 320)
        def _(t):
            zd_v[pl.ds(t * 16, 16)] = zeros16

        def zero_acc():
            @pl.when(sid < 10)
            def _():
                for t in range(5):
                    pltpu.sync_copy(
                        zb_v, acc_sh.at[pl.ds(sid * 1000 + t * 200, 200), :])

        zero_acc()
        pltpu.sync_copy(zd_v.at[pl.ds(0, 5000)], den_sh.at[pl.ds(sid * 5000, 5000)])
        plsc.subcore_barrier()

        lane = lax.iota(jnp.int32, 16)
        rloc = lane // 2
        cloc = lane - rloc * 2

        for qq in range(2):
            qg = cid * 2 + qq
            row_off = qg * n

            @pl.loop(0, n_chunks // 2)
            def _(kp):
                for b in range(2):
                    kk = kp * 2 + b
                    base = sid * per_sub + kk * _C
                    didx_v, didx16_v = didx_b[b], didx16_b[b]
                    ex_v, h_v = ex_b[b], h_b[b]

                    @pl.when(kk >= 2)
                    def _():
                        pltpu.make_async_copy(
                            ex_v, den_sh.at[didx16_v], es_b[b]).wait()
                        pltpu.make_async_copy(
                            h_v, acc_sh.at[didx_v], hs_b[b]).wait()

                    pltpu.sync_copy(s_hbm.at[pl.ds(base, _C)], sidx_v)
                    pltpu.sync_copy(d_hbm.at[pl.ds(base, _C)], didx_v)

                    @pl.loop(0, _C // 16)
                    def _(t):
                        s2_v[pl.ds(t * 16, 16)] = (
                            sidx_v[pl.ds(t * 16, 16)] + row_off)

                    pltpu.async_copy(h_hbm.at[s2_v], h_v, hg_sem)
                    pltpu.sync_copy(as_hbm.at[sidx_v], as_v)
                    pltpu.sync_copy(ad_hbm.at[didx_v], ad_v)

                    @pl.loop(0, _C // 8)
                    def _(q):
                        r = q * 8 + rloc
                        c = cloc + qg * 2
                        al = (plsc.load_gather(as_v, [r, c])
                              + plsc.load_gather(ad_v, [r, c]))
                        al = jnp.maximum(al, _NEG_SLOPE * al)
                        ex_v[pl.ds(q * 16, 16)] = jnp.exp(al)
                        dv = plsc.load_gather(didx_v, [r])
                        didx16_v[pl.ds(q * 16, 16)] = dv * 8 + (cloc + qq * 2)

                    pltpu.make_async_copy(h_hbm.at[s2_v], h_v, hg_sem).wait()

                    @pl.loop(0, _C)
                    def _(i):
                        for j2 in range(2):
                            idxv = jnp.full((16,), 0, jnp.int32) + (i * 2 + j2)
                            sc = plsc.load_gather(ex_v, [idxv])
                            for half in range(2):
                                sl = pl.ds(j2 * 32 + half * 16, 16)
                                h_v[i, sl] = h_v[i, sl] * sc

                    pltpu.async_copy(ex_v, den_sh.at[didx16_v], es_b[b],
                                     add=True)
                    pltpu.async_copy(h_v, acc_sh.at[didx_v], hs_b[b],
                                     add=True)

            for b in range(2):
                pltpu.make_async_copy(ex_b[b], den_sh.at[didx16_b[b]],
                                      es_b[b]).wait()
                pltpu.make_async_copy(h_b[b], acc_sh.at[didx_b[b]],
                                      hs_b[b]).wait()
            plsc.subcore_barrier()

            @pl.when(sid < 10)
            def _():
                pltpu.sync_copy(acc_sh.at[pl.ds(sid * 1000, 1000), :],
                                acc_hbm.at[pl.ds(row_off + sid * 1000, 1000), :])

            if qq == 0:
                zero_acc()
                plsc.subcore_barrier()

        pltpu.sync_copy(den_sh.at[pl.ds(sid * 5000, 5000)],
                        den_hbm.at[pl.ds(cid * n * 8 + sid * 5000, 5000)])

    acc, den = k(h4, a_src, a_dst, s_idx, d_idx)
    return acc, den


def kernel(x_author, x_paper, edge_index_writes, edge_index_written_by, params):
    sw, dw = edge_index_writes[0], edge_index_writes[1]
    swb, dwb = edge_index_written_by[0], edge_index_written_by[1]

    state_a = x_author
    state_p = x_paper
    for l, p in enumerate(params):
        fused = l > 0
        ha4, a_s_w, a_d_wb = _project(
            state_a, p["proj_w"]["author"], p["proj_b"]["author"],
            p["lin_src"]["author__writes__paper"],
            p["lin_dst"]["paper__written_by__author"],
            fused_norm=fused,
        )
        hp4, a_s_wb, a_d_w = _project(
            state_p, p["proj_w"]["paper"], p["proj_b"]["paper"],
            p["lin_src"]["paper__written_by__author"],
            p["lin_dst"]["author__writes__paper"],
            fused_norm=fused,
        )
        acc_p, den_p = _sc_edge(ha4, a_s_w, a_d_w, sw, dw)
        acc_a, den_a = _sc_edge(hp4, a_s_wb, a_d_wb, swb, dwb)
        state_a = (acc_a, den_a)
        state_p = (acc_p, den_p)

    out_a = _finalize(*state_a)
    out_p = _finalize(*state_p)
    return out_a, out_p


# speedup vs baseline: 43.1675x; 1.1364x over previous
import dataclasses
import functools

import jax
import jax.numpy as jnp
from jax import lax
from jax.experimental import pallas as pl
from jax.experimental.pallas import tpu as pltpu
from jax.experimental.pallas import tpu_sc as plsc

_HEADS = 8
_DIM = 32
_HIDDEN = 256
_NEG_SLOPE = 0.2
_ROW_BLOCK = 1000
_C = 400
_NSUB = 16
_EPS = 1e-16


def _att_coefs(h, ls, ld):
    hr = h.reshape(h.shape[0], _HEADS, _DIM)
    return (hr * ls[None]).sum(-1), (hr * ld[None]).sum(-1)


def _proj_body(x_ref, w_ref, b_ref, ls_ref, ld_ref, h4_ref, as_ref, ad_ref):
    h = jnp.dot(x_ref[...], w_ref[...], preferred_element_type=jnp.float32)
    h = h + b_ref[...]
    as_ref[...], ad_ref[...] = _att_coefs(h, ls_ref[...], ld_ref[...])
    h4_ref[...] = jnp.swapaxes(h.reshape(h.shape[0], 4, 64), 0, 1)


def _unnormalize(acc_ref, den_ref):
    rb = acc_ref.shape[1]
    xs = []
    for q in range(4):
        dpair = den_ref[q // 2][:, (q % 2) * 2:(q % 2) * 2 + 2]
        scale = 1.0 / (dpair + _EPS)
        scale = jnp.broadcast_to(scale[:, :, None], (rb, 2, _DIM))
        xs.append(jax.nn.relu(acc_ref[q]) * scale.reshape(rb, 64))
    return jnp.concatenate(xs, axis=1)


def _norm_proj_body(acc_ref, den_ref, w_ref, b_ref, ls_ref, ld_ref,
                    h4_ref, as_ref, ad_ref):
    x = _unnormalize(acc_ref, den_ref)
    h = jnp.dot(x, w_ref[...], preferred_element_type=jnp.float32)
    h = h + b_ref[...]
    as_ref[...], ad_ref[...] = _att_coefs(h, ls_ref[...], ld_ref[...])
    h4_ref[...] = jnp.swapaxes(h.reshape(x.shape[0], 4, 64), 0, 1)


def _final_body(acc_ref, den_ref, o_ref):
    o_ref[...] = _unnormalize(acc_ref, den_ref)


def _project(x_or_accden, w, b, lin_src, lin_dst, *, fused_norm):
    n = 10000
    grid = (n // _ROW_BLOCK,)
    out_specs = [
        pl.BlockSpec((4, _ROW_BLOCK, 64), lambda i: (0, i, 0)),
        pl.BlockSpec((_ROW_BLOCK, _HEADS), lambda i: (i, 0)),
        pl.BlockSpec((_ROW_BLOCK, _HEADS), lambda i: (i, 0)),
    ]
    out_shape = [
        jax.ShapeDtypeStruct((4, n, 64), jnp.float32),
        jax.ShapeDtypeStruct((n, _HEADS), jnp.float32),
        jax.ShapeDtypeStruct((n, _HEADS), jnp.float32),
    ]
    w_specs = [
        pl.BlockSpec((w.shape[0], _HIDDEN), lambda i: (0, 0)),
        pl.BlockSpec((1, _HIDDEN), lambda i: (0, 0)),
        pl.BlockSpec((_HEADS, _DIM), lambda i: (0, 0)),
        pl.BlockSpec((_HEADS, _DIM), lambda i: (0, 0)),
    ]
    if fused_norm:
        acc, den = x_or_accden
        h4, a_s, a_d = pl.pallas_call(
            _norm_proj_body,
            grid=grid,
            in_specs=[
                pl.BlockSpec((4, _ROW_BLOCK, 64), lambda i: (0, i, 0)),
                pl.BlockSpec((2, _ROW_BLOCK, 8), lambda i: (0, i, 0)),
            ] + w_specs,
            out_specs=out_specs,
            out_shape=out_shape,
        )(acc.reshape(4, n, 64), den.reshape(2, n, 8),
          w, b.reshape(1, _HIDDEN), lin_src, lin_dst)
    else:
        x = x_or_accden
        h4, a_s, a_d = pl.pallas_call(
            _proj_body,
            grid=grid,
            in_specs=[
                pl.BlockSpec((_ROW_BLOCK, x.shape[1]), lambda i: (i, 0)),
            ] + w_specs,
            out_specs=out_specs,
            out_shape=out_shape,
        )(x, w, b.reshape(1, _HIDDEN), lin_src, lin_dst)
    return h4.reshape(4 * n, 64), a_s, a_d


def _finalize(acc, den):
    n = 10000
    grid = (n // _ROW_BLOCK,)
    return pl.pallas_call(
        _final_body,
        grid=grid,
        in_specs=[
            pl.BlockSpec((4, _ROW_BLOCK, 64), lambda i: (0, i, 0)),
            pl.BlockSpec((2, _ROW_BLOCK, 8), lambda i: (0, i, 0)),
        ],
        out_specs=pl.BlockSpec((_ROW_BLOCK, _HIDDEN), lambda i: (i, 0)),
        out_shape=jax.ShapeDtypeStruct((n, _HIDDEN), jnp.float32),
    )(acc.reshape(4, n, 64), den.reshape(2, n, 8))


def _sc_edge(h4, a_src, a_dst, s_idx, d_idx):
    n = a_src.shape[0]
    e = s_idx.shape[0]
    per_sub = e // _NSUB
    n_chunks = per_sub // _C

    mesh = plsc.VectorSubcoreMesh(core_axis_name="c", subcore_axis_name="s")
    cp = pltpu.CompilerParams()
    if "needs_layout_passes" in pltpu.CompilerParams.__dataclass_fields__:
        cp = dataclasses.replace(cp, needs_layout_passes=False)
    if "use_tc_tiling_on_sc" in pltpu.CompilerParams.__dataclass_fields__:
        cp = dataclasses.replace(cp, use_tc_tiling_on_sc=False)

    @functools.partial(
        pl.kernel,
        compiler_params=cp,
        out_type=[
            jax.ShapeDtypeStruct((4 * n, 64), jnp.float32),
            jax.ShapeDtypeStruct((2 * n * 8,), jnp.float32),
        ],
        mesh=mesh,
        scratch_types=[
            pltpu.VMEM((_C,), jnp.int32),
            pltpu.VMEM((_C,), jnp.int32),
            pltpu.VMEM((_C,), jnp.int32),
            pltpu.VMEM((_C,), jnp.int32),
            pltpu.VMEM((2 * _C,), jnp.int32),
            pltpu.VMEM((2 * _C,), jnp.int32),
            pltpu.VMEM((_C, 8), jnp.float32),
            pltpu.VMEM((_C, 8), jnp.float32),
            pltpu.VMEM((2 * _C,), jnp.float32),
            pltpu.VMEM((2 * _C,), jnp.float32),
            pltpu.VMEM((_C, 64), jnp.float32),
            pltpu.VMEM((_C, 64), jnp.float32),
            pltpu.VMEM((200, 64), jnp.float32),
            pltpu.VMEM((5120,), jnp.float32),
            pltpu.VMEM_SHARED((n, 64), jnp.float32),
            pltpu.VMEM_SHARED((n * 8,), jnp.float32),
            pltpu.SemaphoreType.DMA,
            pltpu.SemaphoreType.DMA,
            pltpu.SemaphoreType.DMA,
            pltpu.SemaphoreType.DMA,
            pltpu.SemaphoreType.DMA,
            pltpu.SemaphoreType.DMA,
            pltpu.SemaphoreType.DMA,
            pltpu.SemaphoreType.DMA,
        ],
    )
    def k(h_hbm, as_hbm, ad_hbm, s_hbm, d_hbm, acc_hbm, den_hbm,
          sidx_v, didx_v0, didx_v1, s2_v, didx16_v0, didx16_v1,
          as_v, ad_v, ex_v0, ex_v1, h_v0, h_v1,
          zb_v, zd_v, acc_sh, den_sh, hg_sem, ig_sem, ag_sem, ad_sem,
          es0, es1, hs0, hs1):
        didx_b = [didx_v0, didx_v1]
        didx16_b = [didx16_v0, didx16_v1]
        ex_b = [ex_v0, ex_v1]
        h_b = [h_v0, h_v1]
        es_b = [es0, es1]
        hs_b = [hs0, hs1]
        cid = lax.axis_index("c")
        sid = lax.axis_index("s")
        zeros16 = jnp.full((16,), 0.0, jnp.float32)
        @pl.loop(0, 200)
        def _(r):
            for t in range(4):
                zb_v[r, pl.ds(t * 16, 16)] = zeros16

        @pl.loop(0, 320)
        def _(t):
            zd_v[pl.ds(t * 16, 16)] = zeros16

        def zero_acc():
            @pl.when(sid < 10)
            def _():
                for t in range(5):
                    pltpu.sync_copy(
                        zb_v, acc_sh.at[pl.ds(sid * 1000 + t * 200, 200), :])

        zero_acc()
        pltpu.sync_copy(zd_v.at[pl.ds(0, 5000)], den_sh.at[pl.ds(sid * 5000, 5000)])
        plsc.subcore_barrier()

        lane = lax.iota(jnp.int32, 16)
        rloc = lane // 2
        cloc = lane - rloc * 2

        for qq in range(2):
            qg = cid * 2 + qq
            row_off = qg * n

            @pl.loop(0, n_chunks // 2)
            def _(kp):
                for b in range(2):
                    kk = kp * 2 + b
                    base = sid * per_sub + kk * _C
                    didx_v, didx16_v = didx_b[b], didx16_b[b]
                    ex_v, h_v = ex_b[b], h_b[b]

                    @pl.when(kk >= 2)
                    def _():
                        pltpu.make_async_copy(
                            ex_v, den_sh.at[didx16_v], es_b[b]).wait()
                        pltpu.make_async_copy(
                            h_v, acc_sh.at[didx_v], hs_b[b]).wait()

                    pltpu.async_copy(s_hbm.at[pl.ds(base, _C)], sidx_v, ig_sem)
                    pltpu.async_copy(d_hbm.at[pl.ds(base, _C)], didx_v, ad_sem)
                    pltpu.make_async_copy(
                        s_hbm.at[pl.ds(base, _C)], sidx_v, ig_sem).wait()

                    @pl.loop(0, _C // 16)
                    def _(t):
                        s2_v[pl.ds(t * 16, 16)] = (
                            sidx_v[pl.ds(t * 16, 16)] + row_off)

                    pltpu.async_copy(h_hbm.at[s2_v], h_v, hg_sem)
                    pltpu.async_copy(as_hbm.at[sidx_v], as_v, ag_sem)
                    pltpu.make_async_copy(
                        d_hbm.at[pl.ds(base, _C)], didx_v, ad_sem).wait()
                    pltpu.async_copy(ad_hbm.at[didx_v], ad_v, ad_sem)
                    pltpu.make_async_copy(as_hbm.at[sidx_v], as_v, ag_sem).wait()
                    pltpu.make_async_copy(ad_hbm.at[didx_v], ad_v, ad_sem).wait()

                    @pl.loop(0, _C // 8)
                    def _(q):
                        r = q * 8 + rloc
                        c = cloc + qg * 2
                        al = (plsc.load_gather(as_v, [r, c])
                              + plsc.load_gather(ad_v, [r, c]))
                        al = jnp.maximum(al, _NEG_SLOPE * al)
                        ex_v[pl.ds(q * 16, 16)] = jnp.exp(al)
                        dv = plsc.load_gather(didx_v, [r])
                        didx16_v[pl.ds(q * 16, 16)] = dv * 8 + (cloc + qq * 2)

                    pltpu.make_async_copy(h_hbm.at[s2_v], h_v, hg_sem).wait()

                    @pl.loop(0, _C)
                    def _(i):
                        for j2 in range(2):
                            idxv = jnp.full((16,), 0, jnp.int32) + (i * 2 + j2)
                            sc = plsc.load_gather(ex_v, [idxv])
                            for half in range(2):
                                sl = pl.ds(j2 * 32 + half * 16, 16)
                                h_v[i, sl] = h_v[i, sl] * sc

                    pltpu.async_copy(ex_v, den_sh.at[didx16_v], es_b[b],
                                     add=True)
                    pltpu.async_copy(h_v, acc_sh.at[didx_v], hs_b[b],
                                     add=True)

            for b in range(2):
                pltpu.make_async_copy(ex_b[b], den_sh.at[didx16_b[b]],
                                      es_b[b]).wait()
                pltpu.make_async_copy(h_b[b], acc_sh.at[didx_b[b]],
                                      hs_b[b]).wait()
            plsc.subcore_barrier()

            @pl.when(sid < 10)
            def _():
                pltpu.sync_copy(acc_sh.at[pl.ds(sid * 1000, 1000), :],
                                acc_hbm.at[pl.ds(row_off + sid * 1000, 1000), :])

            if qq == 0:
                zero_acc()
                plsc.subcore_barrier()

        pltpu.sync_copy(den_sh.at[pl.ds(sid * 5000, 5000)],
                        den_hbm.at[pl.ds(cid * n * 8 + sid * 5000, 5000)])

    acc, den = k(h4, a_src, a_dst, s_idx, d_idx)
    return acc, den


def kernel(x_author, x_paper, edge_index_writes, edge_index_written_by, params):
    sw, dw = edge_index_writes[0], edge_index_writes[1]
    swb, dwb = edge_index_written_by[0], edge_index_written_by[1]

    state_a = x_author
    state_p = x_paper
    for l, p in enumerate(params):
        fused = l > 0
        ha4, a_s_w, a_d_wb = _project(
            state_a, p["proj_w"]["author"], p["proj_b"]["author"],
            p["lin_src"]["author__writes__paper"],
            p["lin_dst"]["paper__written_by__author"],
            fused_norm=fused,
        )
        hp4, a_s_wb, a_d_w = _project(
            state_p, p["proj_w"]["paper"], p["proj_b"]["paper"],
            p["lin_src"]["paper__written_by__author"],
            p["lin_dst"]["author__writes__paper"],
            fused_norm=fused,
        )
        acc_p, den_p = _sc_edge(ha4, a_s_w, a_d_w, sw, dw)
        acc_a, den_a = _sc_edge(hp4, a_s_wb, a_d_wb, swb, dwb)
        state_a = (acc_a, den_a)
        state_p = (acc_p, den_p)

    out_a = _finalize(*state_a)
    out_p = _finalize(*state_p)
    return out_a, out_p
